# baseline (speedup 1.0000x reference)
import jax
import jax.numpy as jnp
from jax import lax
from jax.experimental import pallas as pl
from jax.experimental.pallas import tpu as pltpu
from jax.experimental.pallas import tpu_sc as plsc

TOPK = 8
NUM_WORKERS = 32


def _stats_kernel(x_ref, pk_ref, sim_ref, idx_ref, xn_ref, pn_ref, rs_ref,
                  means_ref):
    i = pl.program_id(0)
    rows = x_ref.shape[0]
    n = x_ref.shape[1]
    p = pk_ref.shape[0]
    b = means_ref.shape[0]

    x = x_ref[...]
    means_ref[pl.ds(i * rows, rows), :] = jnp.sum(x, axis=1) / jnp.float32(n)

    @pl.when(i == pl.num_programs(0) - 1)
    def _tail():
        pk = pk_ref[...]
        pss = jnp.sum(pk * pk, axis=1, keepdims=True)
        pn = pk * lax.rsqrt(jnp.maximum(pss, jnp.float32(1e-12)))
        pn_ref[...] = pn

        mm = means_ref[...]
        mss = jnp.sum(mm * mm, axis=1, keepdims=True)
        xn = mm * lax.rsqrt(jnp.maximum(mss, jnp.float32(1e-12)))
        xn_ref[...] = xn

        sim = lax.dot_general(
            xn, pn, (((1,), (1,)), ((), ())),
            precision=lax.Precision.DEFAULT,
            preferred_element_type=jnp.float32)
        sim_ref[...] = sim

        iota = lax.broadcasted_iota(jnp.int32, (b, p), 1)
        kiota = lax.broadcasted_iota(jnp.int32, (b, TOPK), 1)
        vals = sim
        idx_acc = jnp.zeros((b, TOPK), jnp.int32)
        ssum = jnp.float32(0.0)
        for k in range(TOPK):
            m = jnp.max(vals, axis=1, keepdims=True)
            im = jnp.min(jnp.where(vals == m, iota, jnp.int32(p)),
                         axis=1, keepdims=True)
            idx_acc = jnp.where(kiota == k, im, idx_acc)
            ssum = ssum + jnp.sum(m)
            vals = jnp.where(iota == im, -jnp.inf, vals)
        idx_ref[...] = idx_acc
        rs_ref[...] = jnp.full((1, 1), ssum / jnp.float32(b), jnp.float32)


def _make_assemble(b, n, c, p, length):
    bpw = b // NUM_WORKERS
    out_rows = TOPK * length + n

    mesh = plsc.VectorSubcoreMesh(core_axis_name="c", subcore_axis_name="s",
                                  num_cores=2, num_subcores=16)

    def body(x_hbm, prompt_hbm, pn_hbm, idx_hbm, out_hbm, bkn_hbm):
        wid = lax.axis_index("s") * 2 + lax.axis_index("c")
        base = wid * bpw

        def gather_phase(gbuf, bknbuf, idxv, gsem, wsem):
            for j in range(bpw):
                bb = base + j
                xw = pltpu.async_copy(
                    x_hbm.at[bb],
                    out_hbm.at[bb, pl.ds(TOPK * length, n), :], wsem)
                pltpu.sync_copy(idx_hbm.at[bb], idxv)
                pltpu.async_copy(prompt_hbm.at[idxv], gbuf, gsem).wait()
                pltpu.async_copy(pn_hbm.at[idxv], bknbuf, gsem).wait()
                waits = [
                    pltpu.async_copy(
                        gbuf.at[r],
                        out_hbm.at[bb, pl.ds(r * length, length), :], wsem)
                    for r in range(TOPK)
                ]
                waits.append(pltpu.async_copy(bknbuf, bkn_hbm.at[bb], wsem))
                waits.append(xw)
                for w in waits:
                    w.wait()

        pl.run_scoped(gather_phase,
                      pltpu.VMEM((TOPK, length, c), jnp.float32),
                      pltpu.VMEM((TOPK, c), jnp.float32),
                      pltpu.VMEM((TOPK,), jnp.int32),
                      pltpu.SemaphoreType.DMA,
                      pltpu.SemaphoreType.DMA)

    return pl.kernel(
        body,
        out_type=(
            jax.ShapeDtypeStruct((b, out_rows, c), jnp.float32),
            jax.ShapeDtypeStruct((b, TOPK, c), jnp.float32),
        ),
        mesh=mesh,
        compiler_params=pltpu.CompilerParams(use_tc_tiling_on_sc=False),
    )


def kernel(x_embed, prompt_key, prompt):
    b, n, c = x_embed.shape
    p = prompt_key.shape[0]
    length = prompt.shape[1]
    out_rows = TOPK * length + n
    chunk = b // 8

    in_specs = [
        pl.BlockSpec((chunk, n, c), lambda i: (i, 0, 0)),
        pl.BlockSpec((p, c), lambda i: (0, 0)),
    ]
    out_shapes = (
        jax.ShapeDtypeStruct((b, p), jnp.float32),
        jax.ShapeDtypeStruct((b, TOPK), jnp.int32),
        jax.ShapeDtypeStruct((b, c), jnp.float32),
        jax.ShapeDtypeStruct((p, c), jnp.float32),
        jax.ShapeDtypeStruct((1, 1), jnp.float32),
    )
    out_specs = (
        pl.BlockSpec((b, p), lambda i: (0, 0)),
        pl.BlockSpec((b, TOPK), lambda i: (0, 0)),
        pl.BlockSpec((b, c), lambda i: (0, 0)),
        pl.BlockSpec((p, c), lambda i: (0, 0)),
        pl.BlockSpec((1, 1), lambda i: (0, 0)),
    )
    sim, idx, xn, pn, rs = pl.pallas_call(
        _stats_kernel,
        grid=(b // chunk,),
        in_specs=in_specs,
        out_specs=out_specs,
        out_shape=out_shapes,
        scratch_shapes=[pltpu.VMEM((b, c), jnp.float32)],
    )(x_embed, prompt_key)

    assemble = _make_assemble(b, n, c, p, length)
    prompted, bkn = assemble(x_embed, prompt, pn, idx)

    return (prompted,
            sim,
            rs.reshape(()),
            idx,
            pn,
            xn,
            bkn)


# speedup vs baseline: 1.1013x; 1.1013x over previous
import jax
import jax.numpy as jnp
from jax import lax
from jax.experimental import pallas as pl
from jax.experimental.pallas import tpu as pltpu
from jax.experimental.pallas import tpu_sc as plsc

TOPK = 8
NUM_WORKERS = 32


def _stats_kernel(x_ref, pk_ref, sim_ref, idx_ref, xn_ref,
                  pn_ref, rs_ref, means_ref):
    i = pl.program_id(0)
    rows = x_ref.shape[0]
    n = x_ref.shape[1]
    p = pk_ref.shape[0]
    b = means_ref.shape[0]

    x = x_ref[...]
    means_ref[pl.ds(i * rows, rows), :] = jnp.sum(x, axis=1) / jnp.float32(n)

    @pl.when(i == pl.num_programs(0) - 1)
    def _tail():
        pk = pk_ref[...]
        pss = jnp.sum(pk * pk, axis=1, keepdims=True)
        pn = pk * lax.rsqrt(jnp.maximum(pss, jnp.float32(1e-12)))
        pn_ref[...] = pn

        mm = means_ref[...]
        mss = jnp.sum(mm * mm, axis=1, keepdims=True)
        xn = mm * lax.rsqrt(jnp.maximum(mss, jnp.float32(1e-12)))
        xn_ref[...] = xn

        sim = lax.dot_general(
            xn, pn, (((1,), (1,)), ((), ())),
            precision=lax.Precision.DEFAULT,
            preferred_element_type=jnp.float32)
        sim_ref[...] = sim

        iota = lax.broadcasted_iota(jnp.int32, (b, p), 1)
        kiota = lax.broadcasted_iota(jnp.int32, (b, TOPK), 1)
        vals = sim
        idx_acc = jnp.zeros((b, TOPK), jnp.int32)
        ssum = jnp.float32(0.0)
        for k in range(TOPK):
            m = jnp.max(vals, axis=1, keepdims=True)
            im = jnp.min(jnp.where(vals == m, iota, jnp.int32(p)),
                         axis=1, keepdims=True)
            idx_acc = jnp.where(kiota == k, im, idx_acc)
            ssum = ssum + jnp.sum(m)
            vals = jnp.where(iota == im, -jnp.inf, vals)
        idx_ref[...] = idx_acc
        rs_ref[...] = jnp.full((1, 1), ssum / jnp.float32(b), jnp.float32)


def _make_assemble(b, n, c, p, length):
    bpw = b // NUM_WORKERS
    out_rows = TOPK * length + n
    grows = TOPK * length
    xal = (n // 8) * 8

    mesh = plsc.VectorSubcoreMesh(core_axis_name="c", subcore_axis_name="s",
                                  num_cores=2, num_subcores=16)

    lanes = 16

    def body(x_hbm, prompt_hbm, pn_hbm, idx_hbm, kkm_hbm, rrm_hbm, kcm_hbm,
             out_hbm, bkn_hbm):
        wid = lax.axis_index("s") * 2 + lax.axis_index("c")
        base = wid * bpw

        def gather_phase(gbuf, bknbuf, idxv2, idxe, idxb, kkv, rrv, kcv,
                         gsem, wsem):
            pltpu.sync_copy(kkm_hbm, kkv)
            pltpu.sync_copy(rrm_hbm, rrv)
            pltpu.sync_copy(kcm_hbm, kcv)
            pltpu.sync_copy(idx_hbm.at[pl.ds(base, bpw), :], idxv2)
            for j in range(bpw):
                bb = base + j
                xw = pltpu.async_copy(
                    x_hbm.at[bb, pl.ds(0, xal), :],
                    out_hbm.at[bb, pl.ds(grows, xal), :], wsem)
                jv = jnp.broadcast_to(jnp.int32(j), (lanes,))
                for ch in range(grows // lanes):
                    sl = pl.ds(ch * lanes, lanes)
                    vals = plsc.load_gather(idxv2, [jv, kkv[sl]])
                    idxe[sl] = vals * jnp.int32(length) + rrv[sl]
                idxb[...] = plsc.load_gather(idxv2, [jv, kcv[...]])
                pltpu.async_copy(prompt_hbm.at[idxe], gbuf, gsem).wait()
                pltpu.async_copy(
                    pn_hbm.at[idxb.at[pl.ds(0, TOPK)]], bknbuf, gsem).wait()
                gw = pltpu.async_copy(
                    gbuf, out_hbm.at[bb, pl.ds(0, grows), :], wsem)
                bw = pltpu.async_copy(bknbuf, bkn_hbm.at[bb], wsem)
                gw.wait()
                bw.wait()
                xw.wait()

        pl.run_scoped(gather_phase,
                      pltpu.VMEM((grows, c), jnp.float32),
                      pltpu.VMEM((TOPK, c), jnp.float32),
                      pltpu.VMEM((bpw, TOPK), jnp.int32),
                      pltpu.VMEM((grows,), jnp.int32),
                      pltpu.VMEM((lanes,), jnp.int32),
                      pltpu.VMEM((grows,), jnp.int32),
                      pltpu.VMEM((grows,), jnp.int32),
                      pltpu.VMEM((lanes,), jnp.int32),
                      pltpu.SemaphoreType.DMA,
                      pltpu.SemaphoreType.DMA)

    return pl.kernel(
        body,
        out_type=(
            jax.ShapeDtypeStruct((b, out_rows, c), jnp.float32),
            jax.ShapeDtypeStruct((b, TOPK, c), jnp.float32),
        ),
        mesh=mesh,
        compiler_params=pltpu.CompilerParams(needs_layout_passes=False),
    )


def kernel(x_embed, prompt_key, prompt):
    b, n, c = x_embed.shape
    p = prompt_key.shape[0]
    length = prompt.shape[1]
    out_rows = TOPK * length + n
    chunk = b // 8

    in_specs = [
        pl.BlockSpec((chunk, n, c), lambda i: (i, 0, 0)),
        pl.BlockSpec((p, c), lambda i: (0, 0)),
    ]
    out_shapes = (
        jax.ShapeDtypeStruct((b, p), jnp.float32),
        jax.ShapeDtypeStruct((b, TOPK), jnp.int32),
        jax.ShapeDtypeStruct((b, c), jnp.float32),
        jax.ShapeDtypeStruct((p, c), jnp.float32),
        jax.ShapeDtypeStruct((1, 1), jnp.float32),
    )
    out_specs = (
        pl.BlockSpec((b, p), lambda i: (0, 0)),
        pl.BlockSpec((b, TOPK), lambda i: (0, 0)),
        pl.BlockSpec((b, c), lambda i: (0, 0)),
        pl.BlockSpec((p, c), lambda i: (0, 0)),
        pl.BlockSpec((1, 1), lambda i: (0, 0)),
    )
    sim, idx, xn, pn, rs = pl.pallas_call(
        _stats_kernel,
        grid=(b // chunk,),
        in_specs=in_specs,
        out_specs=out_specs,
        out_shape=out_shapes,
        scratch_shapes=[pltpu.VMEM((b, c), jnp.float32)],
    )(x_embed, prompt_key)

    grows0 = TOPK * length
    kkm = jnp.arange(grows0, dtype=jnp.int32) // jnp.int32(length)
    rrm = jnp.arange(grows0, dtype=jnp.int32) % jnp.int32(length)
    kcm = jnp.arange(16, dtype=jnp.int32) % jnp.int32(TOPK)
    assemble = _make_assemble(b, n, c, p, length)
    outa, bkn = assemble(x_embed, prompt.reshape(p * length, c), pn, idx,
                         kkm, rrm, kcm)

    xal = (n // 8) * 8
    grows = TOPK * length
    prompted = lax.dynamic_update_slice(
        outa, x_embed[:, xal:, :], (0, grows + xal, 0))

    return (prompted,
            sim,
            rs.reshape(()),
            idx,
            pn,
            xn,
            bkn)


# speedup vs baseline: 8.4211x; 7.6462x over previous
import jax
import jax.numpy as jnp
from jax import lax
from jax.experimental import pallas as pl
from jax.experimental.pallas import tpu as pltpu
from jax.experimental.pallas import tpu_sc as plsc

TOPK = 8
NUM_WORKERS = 32


def _stats_kernel(x_ref, pk_ref, sim_ref, idx_ref, xn_ref,
                  pn_ref, rs_ref, means_ref):
    i = pl.program_id(0)
    rows = x_ref.shape[0]
    n = x_ref.shape[1]
    p = pk_ref.shape[0]
    b = means_ref.shape[0]

    x = x_ref[...]
    means_ref[pl.ds(i * rows, rows), :] = jnp.sum(x, axis=1) / jnp.float32(n)

    @pl.when(i == pl.num_programs(0) - 1)
    def _tail():
        pk = pk_ref[...]
        pss = jnp.sum(pk * pk, axis=1, keepdims=True)
        pn = pk * lax.rsqrt(jnp.maximum(pss, jnp.float32(1e-12)))
        pn_ref[...] = pn

        mm = means_ref[...]
        mss = jnp.sum(mm * mm, axis=1, keepdims=True)
        xn = mm * lax.rsqrt(jnp.maximum(mss, jnp.float32(1e-12)))
        xn_ref[...] = xn

        sim = lax.dot_general(
            xn, pn, (((1,), (1,)), ((), ())),
            precision=lax.Precision.DEFAULT,
            preferred_element_type=jnp.float32)
        sim_ref[...] = sim

        iota = lax.broadcasted_iota(jnp.int32, (b, p), 1)
        kiota = lax.broadcasted_iota(jnp.int32, (b, TOPK), 1)
        vals = sim
        idx_acc = jnp.zeros((b, TOPK), jnp.int32)
        ssum = jnp.float32(0.0)
        for k in range(TOPK):
            m = jnp.max(vals, axis=1, keepdims=True)
            im = jnp.min(jnp.where(vals == m, iota, jnp.int32(p)),
                         axis=1, keepdims=True)
            idx_acc = jnp.where(kiota == k, im, idx_acc)
            ssum = ssum + jnp.sum(m)
            vals = jnp.where(iota == im, -jnp.inf, vals)
        idx_ref[...] = idx_acc
        rs_ref[...] = jnp.full((1, 1), ssum / jnp.float32(b), jnp.float32)


def _make_assemble(b, n, c, p, length):
    bpw = b // NUM_WORKERS
    out_rows = TOPK * length + n
    grows = TOPK * length
    xal = (n // 8) * 8

    mesh = plsc.VectorSubcoreMesh(core_axis_name="c", subcore_axis_name="s",
                                  num_cores=2, num_subcores=16)

    lanes = 16

    def body(x_hbm, prompt_hbm, pn_hbm, idx_hbm, kkm_hbm, rrm_hbm, kcm_hbm,
             out_hbm, bkn_hbm):
        wid = lax.axis_index("s") * 2 + lax.axis_index("c")
        base = wid * bpw

        def gather_phase(gbuf, bknbuf, idxv2, idxe, idxb, kkv, rrv, kcv,
                         gsem, wsem):
            pltpu.sync_copy(kkm_hbm, kkv)
            pltpu.sync_copy(rrm_hbm, rrv)
            pltpu.sync_copy(kcm_hbm, kcv)
            pltpu.sync_copy(idx_hbm.at[pl.ds(base, bpw), :], idxv2)
            half = xal // 2
            for j in range(bpw):
                bb = base + j
                jv = jnp.broadcast_to(jnp.int32(j), (lanes,))
                for ch in range(grows // lanes):
                    sl = pl.ds(ch * lanes, lanes)
                    vals = plsc.load_gather(idxv2, [jv, kkv[sl]])
                    idxe[sl] = vals * jnp.int32(length) + rrv[sl]
                idxb[...] = plsc.load_gather(idxv2, [jv, kcv[...]])
                pltpu.async_copy(prompt_hbm.at[idxe], gbuf, gsem).wait()
                pltpu.async_copy(
                    pn_hbm.at[idxb.at[pl.ds(0, TOPK)]], bknbuf, gsem).wait()
                gw = pltpu.async_copy(
                    gbuf, out_hbm.at[bb, pl.ds(0, grows), :], wsem)
                bw = pltpu.async_copy(bknbuf, bkn_hbm.at[bb], wsem)
                gw.wait()
                bw.wait()
                for o in range(0, xal, half):
                    pltpu.sync_copy(x_hbm.at[bb, pl.ds(o, half), :],
                                    gbuf.at[pl.ds(0, half), :])
                    pltpu.sync_copy(gbuf.at[pl.ds(0, half), :],
                                    out_hbm.at[bb, pl.ds(grows + o, half), :])

        pl.run_scoped(gather_phase,
                      pltpu.VMEM((grows, c), jnp.float32),
                      pltpu.VMEM((TOPK, c), jnp.float32),
                      pltpu.VMEM((bpw, TOPK), jnp.int32),
                      pltpu.VMEM((grows,), jnp.int32),
                      pltpu.VMEM((lanes,), jnp.int32),
                      pltpu.VMEM((grows,), jnp.int32),
                      pltpu.VMEM((grows,), jnp.int32),
                      pltpu.VMEM((lanes,), jnp.int32),
                      pltpu.SemaphoreType.DMA,
                      pltpu.SemaphoreType.DMA)

    return pl.kernel(
        body,
        out_type=(
            jax.ShapeDtypeStruct((b, out_rows, c), jnp.float32),
            jax.ShapeDtypeStruct((b, TOPK, c), jnp.float32),
        ),
        mesh=mesh,
        compiler_params=pltpu.CompilerParams(needs_layout_passes=False),
    )


def kernel(x_embed, prompt_key, prompt):
    b, n, c = x_embed.shape
    p = prompt_key.shape[0]
    length = prompt.shape[1]
    out_rows = TOPK * length + n
    chunk = b // 8

    in_specs = [
        pl.BlockSpec((chunk, n, c), lambda i: (i, 0, 0)),
        pl.BlockSpec((p, c), lambda i: (0, 0)),
    ]
    out_shapes = (
        jax.ShapeDtypeStruct((b, p), jnp.float32),
        jax.ShapeDtypeStruct((b, TOPK), jnp.int32),
        jax.ShapeDtypeStruct((b, c), jnp.float32),
        jax.ShapeDtypeStruct((p, c), jnp.float32),
        jax.ShapeDtypeStruct((1, 1), jnp.float32),
    )
    out_specs = (
        pl.BlockSpec((b, p), lambda i: (0, 0)),
        pl.BlockSpec((b, TOPK), lambda i: (0, 0)),
        pl.BlockSpec((b, c), lambda i: (0, 0)),
        pl.BlockSpec((p, c), lambda i: (0, 0)),
        pl.BlockSpec((1, 1), lambda i: (0, 0)),
    )
    sim, idx, xn, pn, rs = pl.pallas_call(
        _stats_kernel,
        grid=(b // chunk,),
        in_specs=in_specs,
        out_specs=out_specs,
        out_shape=out_shapes,
        scratch_shapes=[pltpu.VMEM((b, c), jnp.float32)],
    )(x_embed, prompt_key)

    grows0 = TOPK * length
    kkm = jnp.arange(grows0, dtype=jnp.int32) // jnp.int32(length)
    rrm = jnp.arange(grows0, dtype=jnp.int32) % jnp.int32(length)
    kcm = jnp.arange(16, dtype=jnp.int32) % jnp.int32(TOPK)
    assemble = _make_assemble(b, n, c, p, length)
    outa, bkn = assemble(x_embed, prompt.reshape(p * length, c), pn, idx,
                         kkm, rrm, kcm)

    xal = (n // 8) * 8
    grows = TOPK * length
    prompted = lax.dynamic_update_slice(
        outa, x_embed[:, xal:, :], (0, grows + xal, 0))

    return (prompted,
            sim,
            rs.reshape(()),
            idx,
            pn,
            xn,
            bkn)


# speedup vs baseline: 17.6794x; 2.0994x over previous
import jax
import jax.numpy as jnp
from jax import lax
from jax.experimental import pallas as pl
from jax.experimental.pallas import tpu as pltpu
from jax.experimental.pallas import tpu_sc as plsc

TOPK = 8
NUM_WORKERS = 32


def _stats_kernel(xt_ref, pk_ref, sim_ref, idx_ref, xn_ref, pn_ref, rs_ref,
                  means_ref):
    i = pl.program_id(0)
    rows = xt_ref.shape[1]
    n = xt_ref.shape[0]
    p = pk_ref.shape[0]
    b = means_ref.shape[0]

    x = xt_ref[...]
    means_ref[pl.ds(i * rows, rows), :] = jnp.sum(x, axis=0) / jnp.float32(n)

    @pl.when(i == pl.num_programs(0) - 1)
    def _tail():
        pk = pk_ref[...]
        pss = jnp.sum(pk * pk, axis=1, keepdims=True)
        pn = pk * lax.rsqrt(jnp.maximum(pss, jnp.float32(1e-12)))
        pn_ref[...] = pn

        mm = means_ref[...]
        mss = jnp.sum(mm * mm, axis=1, keepdims=True)
        xn = mm * lax.rsqrt(jnp.maximum(mss, jnp.float32(1e-12)))
        xn_ref[...] = xn

        sim = lax.dot_general(
            xn, pn, (((1,), (1,)), ((), ())),
            precision=lax.Precision.DEFAULT,
            preferred_element_type=jnp.float32)
        sim_ref[...] = sim

        iota = lax.broadcasted_iota(jnp.int32, (b, p), 1)
        kiota = lax.broadcasted_iota(jnp.int32, (b, TOPK), 1)
        vals = sim
        idx_acc = jnp.zeros((b, TOPK), jnp.int32)
        ssum = jnp.float32(0.0)
        for k in range(TOPK):
            m = jnp.max(vals, axis=1, keepdims=True)
            im = jnp.min(jnp.where(vals == m, iota, jnp.int32(p)),
                         axis=1, keepdims=True)
            idx_acc = jnp.where(kiota == k, im, idx_acc)
            ssum = ssum + jnp.sum(m)
            vals = jnp.where(iota == im, -jnp.inf, vals)
        idx_ref[...] = idx_acc
        rs_ref[...] = jnp.full((1, 1), ssum / jnp.float32(b), jnp.float32)


def _make_assemble(b, n, c, p, length):
    grows = TOPK * length
    out_rows = grows + n
    qsz = b // 8
    nq = b // qsz
    lanes = 16
    gpw = grows // NUM_WORKERS
    cpw = -(-n // NUM_WORKERS)
    bpw = b // NUM_WORKERS

    mesh = plsc.VectorSubcoreMesh(core_axis_name="c", subcore_axis_name="s",
                                  num_cores=2, num_subcores=16)

    def body(xt_hbm, prompt_hbm, pn_hbm, idx_hbm, kcm_hbm, out_hbm, bkn_hbm):
        wid = lax.axis_index("s") * 2 + lax.axis_index("c")

        def run(h0, h1, idxv2, idxg, idxb, kcv, gsem, wsem):
            bufs = (h0, h1)
            pltpu.sync_copy(kcm_hbm, kcv)
            pltpu.sync_copy(idx_hbm, idxv2)

            def staged_slab(src_fn, dst_fn):
                pend = [None, None]
                for q in range(nq):
                    buf = bufs[q % 2]
                    if pend[q % 2] is not None:
                        pend[q % 2].wait()
                    pltpu.async_copy(src_fn(q), buf, gsem).wait()
                    pend[q % 2] = pltpu.async_copy(buf, dst_fn(q), wsem)
                pend[0].wait()
                pend[1].wait()

            for i in range(gpw):
                g = wid * gpw + i
                kk = g // jnp.int32(length)
                rr = g - kk * jnp.int32(length)
                for ch in range(b // lanes):
                    bbv = lax.iota(jnp.int32, lanes) + jnp.int32(ch * lanes)
                    kv = jnp.broadcast_to(kk, (lanes,))
                    vals = plsc.load_gather(idxv2, [bbv, kv])
                    idxg[pl.ds(ch * lanes, lanes)] = (
                        rr * jnp.int32(p) + vals)
                staged_slab(
                    lambda q: prompt_hbm.at[idxg.at[pl.ds(q * qsz, qsz)]],
                    lambda q: out_hbm.at[g, pl.ds(q * qsz, qsz), :])

            for i in range(cpw):
                s = jnp.int32(NUM_WORKERS * i) + wid

                @pl.when(s < n)
                def _():
                    staged_slab(
                        lambda q: xt_hbm.at[s, pl.ds(q * qsz, qsz), :],
                        lambda q: out_hbm.at[grows + s,
                                             pl.ds(q * qsz, qsz), :])

            for j in range(bpw):
                bb = wid * bpw + j
                bv = jnp.broadcast_to(bb, (lanes,))
                idxb[...] = plsc.load_gather(idxv2, [bv, kcv[...]])
                pltpu.async_copy(
                    pn_hbm.at[idxb.at[pl.ds(0, TOPK)]],
                    h0.at[pl.ds(0, TOPK), :], gsem).wait()
                pltpu.async_copy(
                    h0.at[pl.ds(0, TOPK), :], bkn_hbm.at[bb], wsem).wait()

        pl.run_scoped(run,
                      pltpu.VMEM((qsz, c), jnp.float32),
                      pltpu.VMEM((qsz, c), jnp.float32),
                      pltpu.VMEM((b, TOPK), jnp.int32),
                      pltpu.VMEM((b,), jnp.int32),
                      pltpu.VMEM((lanes,), jnp.int32),
                      pltpu.VMEM((lanes,), jnp.int32),
                      pltpu.SemaphoreType.DMA,
                      pltpu.SemaphoreType.DMA)

    return pl.kernel(
        body,
        out_type=(
            jax.ShapeDtypeStruct((out_rows, b, c), jnp.float32),
            jax.ShapeDtypeStruct((b, TOPK, c), jnp.float32),
        ),
        mesh=mesh,
        compiler_params=pltpu.CompilerParams(needs_layout_passes=False),
    )


def kernel(x_embed, prompt_key, prompt):
    b, n, c = x_embed.shape
    p = prompt_key.shape[0]
    length = prompt.shape[1]
    chunk = b // 8

    xt = jnp.transpose(x_embed, (1, 0, 2))
    prompt_t = jnp.transpose(prompt, (1, 0, 2))
    prompt2d = prompt_t.reshape(length * p, c)

    in_specs = [
        pl.BlockSpec((n, chunk, c), lambda i: (0, i, 0)),
        pl.BlockSpec((p, c), lambda i: (0, 0)),
    ]
    out_shapes = (
        jax.ShapeDtypeStruct((b, p), jnp.float32),
        jax.ShapeDtypeStruct((b, TOPK), jnp.int32),
        jax.ShapeDtypeStruct((b, c), jnp.float32),
        jax.ShapeDtypeStruct((p, c), jnp.float32),
        jax.ShapeDtypeStruct((1, 1), jnp.float32),
    )
    out_specs = (
        pl.BlockSpec((b, p), lambda i: (0, 0)),
        pl.BlockSpec((b, TOPK), lambda i: (0, 0)),
        pl.BlockSpec((b, c), lambda i: (0, 0)),
        pl.BlockSpec((p, c), lambda i: (0, 0)),
        pl.BlockSpec((1, 1), lambda i: (0, 0)),
    )
    sim, idx, xn, pn, rs = pl.pallas_call(
        _stats_kernel,
        grid=(b // chunk,),
        in_specs=in_specs,
        out_specs=out_specs,
        out_shape=out_shapes,
        scratch_shapes=[pltpu.VMEM((b, c), jnp.float32)],
    )(xt, prompt_key)

    kcm = jnp.arange(16, dtype=jnp.int32) % jnp.int32(TOPK)
    assemble = _make_assemble(b, n, c, p, length)
    outt, bkn = assemble(xt, prompt2d, pn, idx, kcm)
    prompted = jnp.transpose(outt, (1, 0, 2))

    return (prompted,
            sim,
            rs.reshape(()),
            idx,
            pn,
            xn,
            bkn)
